# baseline (speedup 1.0000x reference)
import functools

import jax
import jax.numpy as jnp
from jax import lax
from jax.experimental import pallas as pl
from jax.experimental.pallas import tpu as pltpu
from jax.experimental.pallas import tpu_sc as plsc

BATCH = 4096
SEQ = 200
VOCAB = 1000
BINF = 64

NUM_CORES = 2
NUM_SUBCORES = 16
NUM_WORKERS = NUM_CORES * NUM_SUBCORES

TOTAL = BATCH * SEQ
PER_WORKER = TOTAL // NUM_WORKERS
CHUNK = 256
NUM_CHUNKS = PER_WORKER // CHUNK
GROUPS = CHUNK // 16
NBUF = 2


def _sc_gather(table_flat, idx):
    mesh = plsc.VectorSubcoreMesh(core_axis_name="c", subcore_axis_name="s")

    @functools.partial(
        pl.kernel,
        mesh=mesh,
        compiler_params=pltpu.CompilerParams(
            use_tc_tiling_on_sc=False, needs_layout_passes=False
        ),
        out_type=jax.ShapeDtypeStruct((TOTAL * BINF,), jnp.float32),
        scratch_types=[
            pltpu.VMEM((VOCAB * BINF,), jnp.float32),
            pltpu.VMEM((PER_WORKER,), jnp.int32),
            [pltpu.VMEM((CHUNK * BINF,), jnp.float32) for _ in range(NBUF)],
            [pltpu.SemaphoreType.DMA for _ in range(NBUF)],
        ],
    )
    def body(table_hbm, idx_hbm, out_hbm, table_v, idx_v, bufs, wsems):
        wid = lax.axis_index("s") * NUM_CORES + lax.axis_index("c")
        base = wid * PER_WORKER

        pltpu.sync_copy(table_hbm, table_v)
        pltpu.sync_copy(idx_hbm.at[pl.ds(base, PER_WORKER)], idx_v)

        lane_out = lax.iota(jnp.int32, 16) * BINF

        def out_slice(g):
            off = pl.multiple_of((base + g * CHUNK) * BINF, CHUNK * BINF)
            return out_hbm.at[pl.ds(off, CHUNK * BINF)]

        def expand(g, buf):
            def jbody(j, c):
                ioff = pl.multiple_of(g * CHUNK + j * 16, 16)
                idx16 = idx_v[pl.ds(ioff, 16)]
                rowbase = idx16 * BINF
                outbase = lane_out + j * (16 * BINF)
                for f in range(BINF):
                    vals = plsc.load_gather(table_v, [rowbase + f])
                    plsc.store_scatter(buf, [outbase + f], vals)
                return c

            lax.fori_loop(0, GROUPS, jbody, 0)

        def outer(i, carry):
            for b in range(NBUF):
                g = i * NBUF + b

                @pl.when(g >= NBUF)
                def _():
                    pltpu.make_async_copy(bufs[b], out_slice(g - NBUF), wsems[b]).wait()

                expand(g, bufs[b])
                pltpu.async_copy(bufs[b], out_slice(g), wsems[b])
            return carry

        lax.fori_loop(0, NUM_CHUNKS // NBUF, outer, 0)
        for b in range(NBUF):
            pltpu.make_async_copy(bufs[b], out_slice(NUM_CHUNKS - NBUF + b), wsems[b]).wait()

    return body(table_flat, idx)


def kernel(x, mapping):
    table_flat = mapping.T.reshape(-1)
    idx = x.reshape(-1).astype(jnp.int32)
    out = _sc_gather(table_flat, idx)
    return out.reshape(BATCH, SEQ, BINF)


# speedup vs baseline: 1.2940x; 1.2940x over previous
import functools

import jax
import jax.numpy as jnp
from jax import lax
from jax.experimental import pallas as pl
from jax.experimental.pallas import tpu as pltpu
from jax.experimental.pallas import tpu_sc as plsc

BATCH = 4096
SEQ = 200
VOCAB = 1000
BINF = 64

NUM_CORES = 2
NUM_SUBCORES = 16
NUM_WORKERS = NUM_CORES * NUM_SUBCORES

TOTAL = BATCH * SEQ
PER_WORKER = TOTAL // NUM_WORKERS
CHUNK = 256
NUM_CHUNKS = PER_WORKER // CHUNK
GROUPS = CHUNK // 16
NBUF = 2


def _sc_gather(table_flat, idx):
    mesh = plsc.VectorSubcoreMesh(core_axis_name="c", subcore_axis_name="s")

    @functools.partial(
        pl.kernel,
        mesh=mesh,
        compiler_params=pltpu.CompilerParams(
            use_tc_tiling_on_sc=False, needs_layout_passes=False
        ),
        out_type=jax.ShapeDtypeStruct((TOTAL * BINF,), jnp.float32),
        scratch_types=[
            pltpu.VMEM((VOCAB * BINF,), jnp.float32),
            pltpu.VMEM((PER_WORKER,), jnp.int32),
            [pltpu.VMEM((CHUNK * BINF,), jnp.float32) for _ in range(NBUF)],
            [pltpu.SemaphoreType.DMA for _ in range(NBUF)],
        ],
    )
    def body(table_hbm, idx_hbm, out_hbm, table_v, idx_v, bufs, wsems):
        wid = lax.axis_index("s") * NUM_CORES + lax.axis_index("c")
        base = wid * PER_WORKER

        pltpu.sync_copy(table_hbm, table_v)
        pltpu.sync_copy(idx_hbm.at[pl.ds(base, PER_WORKER)], idx_v)

        lane_out = lax.iota(jnp.int32, 16) * BINF

        def out_slice(g):
            off = pl.multiple_of((base + g * CHUNK) * BINF, CHUNK * BINF)
            return out_hbm.at[pl.ds(off, CHUNK * BINF)]

        def expand(g, buf):
            def jbody(j, c):
                ioff = pl.multiple_of(g * CHUNK + j * 16, 16)
                idx16 = idx_v[pl.ds(ioff, 16)]
                rowbase = idx16 * BINF
                outbase = lane_out + j * (16 * BINF)
                for fb in range(0, BINF, 16):
                    vals = [
                        plsc.load_gather(table_v, [rowbase + f])
                        for f in range(fb, fb + 16)
                    ]
                    for i, f in enumerate(range(fb, fb + 16)):
                        plsc.store_scatter(buf, [outbase + f], vals[i])
                return c

            lax.fori_loop(0, GROUPS, jbody, 0)

        def outer(i, carry):
            for b in range(NBUF):
                g = i * NBUF + b

                @pl.when(g >= NBUF)
                def _():
                    pltpu.make_async_copy(bufs[b], out_slice(g - NBUF), wsems[b]).wait()

                expand(g, bufs[b])
                pltpu.async_copy(bufs[b], out_slice(g), wsems[b])
            return carry

        lax.fori_loop(0, NUM_CHUNKS // NBUF, outer, 0)
        for b in range(NBUF):
            pltpu.make_async_copy(bufs[b], out_slice(NUM_CHUNKS - NBUF + b), wsems[b]).wait()

    return body(table_flat, idx)


def kernel(x, mapping):
    table_flat = mapping.T.reshape(-1)
    idx = x.reshape(-1).astype(jnp.int32)
    out = _sc_gather(table_flat, idx)
    return out.reshape(BATCH, SEQ, BINF)


# speedup vs baseline: 4.0615x; 3.1389x over previous
import functools

import jax
import jax.numpy as jnp
from jax import lax
from jax.experimental import pallas as pl
from jax.experimental.pallas import tpu as pltpu
from jax.experimental.pallas import tpu_sc as plsc

BATCH = 4096
SEQ = 200
VOCAB = 1000
BINF = 64

NUM_CORES = 2
NUM_SUBCORES = 16
NUM_WORKERS = NUM_CORES * NUM_SUBCORES

TOTAL = BATCH * SEQ
PER_WORKER = TOTAL // NUM_WORKERS
CHUNK = 256
NUM_CHUNKS = PER_WORKER // CHUNK
GROUPS = CHUNK // 16
NBUF = 2


def _sc_gather(table_flat, idx):
    mesh = plsc.VectorSubcoreMesh(core_axis_name="c", subcore_axis_name="s")

    @functools.partial(
        pl.kernel,
        mesh=mesh,
        compiler_params=pltpu.CompilerParams(
            use_tc_tiling_on_sc=False, needs_layout_passes=False
        ),
        out_type=jax.ShapeDtypeStruct((TOTAL * BINF,), jnp.float32),
        scratch_types=[
            pltpu.VMEM((VOCAB * BINF,), jnp.float32),
            pltpu.VMEM((PER_WORKER,), jnp.int32),
            [pltpu.VMEM((CHUNK * BINF,), jnp.float32) for _ in range(NBUF)],
            [pltpu.SemaphoreType.DMA for _ in range(NBUF)],
        ],
    )
    def body(table_hbm, idx_hbm, out_hbm, table_v, idx_v, bufs, wsems):
        wid = lax.axis_index("s") * NUM_CORES + lax.axis_index("c")
        base = wid * PER_WORKER

        pltpu.sync_copy(table_hbm, table_v)
        pltpu.sync_copy(idx_hbm.at[pl.ds(base, PER_WORKER)], idx_v)

        def out_slice(g):
            off = pl.multiple_of((base + g * CHUNK) * BINF, CHUNK * BINF)
            return out_hbm.at[pl.ds(off, CHUNK * BINF)]

        def expand(g, buf):
            @functools.partial(plsc.parallel_loop, 0, CHUNK, unroll=4)
            def rbody(r):
                i = idx_v[g * CHUNK + r]
                tb = pl.multiple_of(i * BINF, BINF)
                ob = pl.multiple_of(r * BINF, BINF)
                vals = [table_v[pl.ds(tb + 16 * q, 16)] for q in range(BINF // 16)]
                for q in range(BINF // 16):
                    buf[pl.ds(ob + 16 * q, 16)] = vals[q]

        def outer(i, carry):
            for b in range(NBUF):
                g = i * NBUF + b

                @pl.when(g >= NBUF)
                def _():
                    pltpu.make_async_copy(bufs[b], out_slice(g - NBUF), wsems[b]).wait()

                expand(g, bufs[b])
                pltpu.async_copy(bufs[b], out_slice(g), wsems[b])
            return carry

        lax.fori_loop(0, NUM_CHUNKS // NBUF, outer, 0)
        for b in range(NBUF):
            pltpu.make_async_copy(bufs[b], out_slice(NUM_CHUNKS - NBUF + b), wsems[b]).wait()

    return body(table_flat, idx)


def kernel(x, mapping):
    table_flat = mapping.T.reshape(-1)
    idx = x.reshape(-1).astype(jnp.int32)
    out = _sc_gather(table_flat, idx)
    return out.reshape(BATCH, SEQ, BINF)


# speedup vs baseline: 6.9611x; 1.7139x over previous
import functools

import jax
import jax.numpy as jnp
from jax import lax
from jax.experimental import pallas as pl
from jax.experimental.pallas import tpu as pltpu
from jax.experimental.pallas import tpu_sc as plsc

BATCH = 4096
SEQ = 200
VOCAB = 1000
BINF = 64

NUM_CORES = 2
NUM_SUBCORES = 16
NUM_WORKERS = NUM_CORES * NUM_SUBCORES

TOTAL = BATCH * SEQ
PER_WORKER = TOTAL // NUM_WORKERS
CHUNK = 128
NUM_CHUNKS = PER_WORKER // CHUNK
GROUPS = CHUNK // 16
NBUF = 2


def _sc_gather(table_flat, idx):
    mesh = plsc.VectorSubcoreMesh(core_axis_name="c", subcore_axis_name="s")

    @functools.partial(
        pl.kernel,
        mesh=mesh,
        compiler_params=pltpu.CompilerParams(needs_layout_passes=False),
        out_type=jax.ShapeDtypeStruct((TOTAL, BINF), jnp.float32),
        scratch_types=[
            pltpu.VMEM((VOCAB * BINF,), jnp.float32),
            pltpu.VMEM((PER_WORKER,), jnp.int32),
            [pltpu.VMEM((CHUNK, BINF), jnp.float32) for _ in range(NBUF)],
            [pltpu.SemaphoreType.DMA for _ in range(NBUF)],
        ],
    )
    def body(table_hbm, idx_hbm, out_hbm, table_v, idx_v, bufs, wsems):
        wid = lax.axis_index("s") * NUM_CORES + lax.axis_index("c")
        base = wid * PER_WORKER

        pltpu.sync_copy(table_hbm, table_v)
        pltpu.sync_copy(idx_hbm.at[pl.ds(base, PER_WORKER)], idx_v)

        def out_slice(g):
            off = pl.multiple_of(base + g * CHUNK, CHUNK)
            return out_hbm.at[pl.ds(off, CHUNK)]

        def expand(g, buf):
            @functools.partial(plsc.parallel_loop, 0, CHUNK, unroll=4)
            def rbody(r):
                i = idx_v[g * CHUNK + r]
                tb = pl.multiple_of(i * BINF, BINF)
                vals = [table_v[pl.ds(tb + 16 * q, 16)] for q in range(BINF // 16)]
                for q in range(BINF // 16):
                    buf[r, pl.ds(16 * q, 16)] = vals[q]

        def outer(i, carry):
            for b in range(NBUF):
                g = i * NBUF + b

                @pl.when(g >= NBUF)
                def _():
                    pltpu.make_async_copy(bufs[b], out_slice(g - NBUF), wsems[b]).wait()

                expand(g, bufs[b])
                pltpu.async_copy(bufs[b], out_slice(g), wsems[b])
            return carry

        lax.fori_loop(0, NUM_CHUNKS // NBUF, outer, 0)
        for b in range(NBUF):
            pltpu.make_async_copy(bufs[b], out_slice(NUM_CHUNKS - NBUF + b), wsems[b]).wait()

    return body(table_flat, idx)


def kernel(x, mapping):
    table_flat = mapping.T.reshape(-1)
    idx = x.reshape(-1).astype(jnp.int32)
    out = _sc_gather(table_flat, idx)
    return out.reshape(BATCH, SEQ, BINF)
